# baseline (speedup 1.0000x reference)
import jax
import jax.numpy as jnp
from jax.experimental import pallas as pl
from jax.experimental.pallas import tpu as pltpu
import functools

N_ROWS = 16384
N_COLS = 1000
BLOCK_ROWS = 512


def _ece_kernel(x_ref, t_ref, out_ref, acc_ref):
    i = pl.program_id(0)

    @pl.when(i == 0)
    def _init():
        acc_ref[0] = 0.0
        acc_ref[1] = 0.0

    x = x_ref[...]
    m = jnp.max(x, axis=1, keepdims=True)
    e = jnp.exp(x - m)
    ones = jnp.ones((N_COLS, 128), jnp.float32)
    s = jax.lax.dot_general(e, ones, (((1,), (0,)), ((), ())),
                            preferred_element_type=jnp.float32)[:, :1]
    conf = 1.0 / s[:, 0]

    idx = jax.lax.broadcasted_iota(jnp.int32, x.shape, 1)
    pred = jnp.min(jnp.where(x == m, idx, N_COLS), axis=1)
    acc = (pred == t_ref[...]).astype(jnp.float32)

    acc_ref[0] += jnp.sum(acc)
    acc_ref[1] += jnp.sum(conf)

    @pl.when(i == pl.num_programs(0) - 1)
    def _fini():
        v = jnp.abs(acc_ref[0] - acc_ref[1]) / N_ROWS
        out_ref[...] = jnp.full((1, 1), v, jnp.float32)


@jax.jit
def _ece(output, target):
    grid = N_ROWS // BLOCK_ROWS
    out = pl.pallas_call(
        _ece_kernel,
        grid=(grid,),
        in_specs=[
            pl.BlockSpec((BLOCK_ROWS, N_COLS), lambda i: (i, 0)),
            pl.BlockSpec((BLOCK_ROWS,), lambda i: (i,)),
        ],
        out_specs=pl.BlockSpec((1, 1), lambda i: (0, 0)),
        out_shape=jax.ShapeDtypeStruct((1, 1), jnp.float32),
        scratch_shapes=[pltpu.SMEM((2,), jnp.float32)],
    )(output, target.astype(jnp.int32))
    return out[0, 0]


def kernel(output, target):
    return _ece(output, target)


# speedup vs baseline: 1.1568x; 1.1568x over previous
import jax
import jax.numpy as jnp
from jax.experimental import pallas as pl
from jax.experimental.pallas import tpu as pltpu
import functools

N_ROWS = 16384
N_COLS = 1000
BLOCK_ROWS = 2048


def _ece_kernel(x_ref, t_ref, out_ref, acc_ref):
    i = pl.program_id(0)

    @pl.when(i == 0)
    def _init():
        acc_ref[0] = 0.0
        acc_ref[1] = 0.0

    x = x_ref[...]
    m = jnp.max(x, axis=1, keepdims=True)
    e = jnp.exp(x - m)
    ones = jnp.ones((N_COLS, 128), jnp.float32)
    s = jax.lax.dot_general(e, ones, (((1,), (0,)), ((), ())),
                            preferred_element_type=jnp.float32)[:, :1]
    conf = 1.0 / s[:, 0]

    idx = jax.lax.broadcasted_iota(jnp.int32, x.shape, 1)
    pred = jnp.min(jnp.where(x == m, idx, N_COLS), axis=1)
    acc = (pred == t_ref[...]).astype(jnp.float32)

    acc_ref[0] += jnp.sum(acc)
    acc_ref[1] += jnp.sum(conf)

    @pl.when(i == pl.num_programs(0) - 1)
    def _fini():
        v = jnp.abs(acc_ref[0] - acc_ref[1]) / N_ROWS
        out_ref[...] = jnp.full((1, 1), v, jnp.float32)


@jax.jit
def _ece(output, target):
    grid = N_ROWS // BLOCK_ROWS
    out = pl.pallas_call(
        _ece_kernel,
        grid=(grid,),
        in_specs=[
            pl.BlockSpec((BLOCK_ROWS, N_COLS), lambda i: (i, 0)),
            pl.BlockSpec((BLOCK_ROWS,), lambda i: (i,)),
        ],
        out_specs=pl.BlockSpec((1, 1), lambda i: (0, 0)),
        out_shape=jax.ShapeDtypeStruct((1, 1), jnp.float32),
        scratch_shapes=[pltpu.SMEM((2,), jnp.float32)],
    )(output, target.astype(jnp.int32))
    return out[0, 0]


def kernel(output, target):
    return _ece(output, target)
